# Initial kernel scaffold; baseline (speedup 1.0000x reference)
import jax
import jax.numpy as jnp
from jax.experimental import pallas as pl


def kernel(x, label, prototypes, queue0, queue1):
    raise NotImplementedError("write your pallas kernel here")


# speedup vs baseline: 3.2933x; 3.2933x over previous
import functools

import jax
import jax.numpy as jnp
from jax.experimental import pallas as pl
from jax.experimental.pallas import tpu as pltpu

_TEMP = 0.07
_EPS = 1e-12


def _rca_kernel(x_ref, pr_ref, prT_ref, q0_ref, q1_ref, out_ref, acc_ref,
                *, nsteps, num_p, num_q, batch):
    j = pl.program_id(0)

    @pl.when(j == 0)
    def _init():
        acc_ref[...] = jnp.zeros_like(acc_ref)

    pr = pr_ref[...]
    for b in range(batch):
        xb = x_ref[b]
        sim = jax.lax.dot_general(pr, xb, (((1,), (0,)), ((), ())),
                                  preferred_element_type=jnp.float32)
        row = jax.lax.broadcasted_iota(jnp.int32, sim.shape, 0)
        sim = jnp.where(row < num_p, sim, -jnp.inf)
        mx = jnp.max(sim, axis=0, keepdims=True)
        idx = jnp.min(jnp.where(sim == mx, row, num_p), axis=0, keepdims=True)
        acc_ref[b] += (row == idx).astype(jnp.float32)

    @pl.when(j == nsteps - 1)
    def _loss():
        prT = prT_ref[...]
        q0 = q0_ref[...]
        q1 = q1_ref[...]
        rowq = jax.lax.broadcasted_iota(jnp.int32, (q0.shape[0], 1), 0)
        validq = rowq < num_q
        q0n = q0 / jnp.maximum(
            jnp.sqrt(jnp.sum(q0 * q0, axis=1, keepdims=True)), _EPS)
        q1n = q1 / jnp.maximum(
            jnp.sqrt(jnp.sum(q1 * q1, axis=1, keepdims=True)), _EPS)
        total = jnp.zeros((1, 1), jnp.float32)
        for b in range(batch):
            cnt = jnp.sum(acc_ref[b], axis=1, keepdims=True)
            pool = jax.lax.dot_general(prT, cnt, (((1,), (0,)), ((), ())),
                                       preferred_element_type=jnp.float32)
            n = pool / jnp.maximum(
                jnp.sqrt(jnp.sum(pool * pool, axis=0, keepdims=True)), _EPS)
            s_neg = jax.lax.dot_general(q0n, n, (((1,), (0,)), ((), ())),
                                        preferred_element_type=jnp.float32)
            logit_neg = jnp.where(validq, s_neg / _TEMP, -jnp.inf)
            m = jnp.max(logit_neg, axis=0, keepdims=True)
            eln = jnp.where(validq, jnp.exp(logit_neg - m), 0.0)
            l_neg = jnp.sum(eln, axis=0, keepdims=True)
            s_pos = jax.lax.dot_general(q1n, n, (((1,), (0,)), ((), ())),
                                        preferred_element_type=jnp.float32)
            logit_pos = s_pos / _TEMP - m
            elp = jnp.exp(logit_pos)
            terms = -(logit_pos - jnp.log(jnp.maximum(l_neg + elp, 1e-4)))
            loss_b = jnp.sum(jnp.where(validq, terms, 0.0),
                             axis=0, keepdims=True) / num_q
            total = total + loss_b
        out_ref[...] = total / batch


def kernel(x, label, prototypes, queue0, queue1):
    del label
    B, C = x.shape[0], x.shape[1]
    S = x.shape[2] * x.shape[3] * x.shape[4]
    P = prototypes.shape[0]
    Q = queue0.shape[0]
    PPAD = 16
    QPAD = 16
    CHUNK = 2048
    nsteps = S // CHUNK

    x2 = x.reshape(B, C, S)
    pr = jnp.zeros((PPAD, C), jnp.float32).at[:P].set(
        prototypes.reshape(P, C).astype(jnp.float32))
    prT = pr.T
    q0 = jnp.zeros((QPAD, C), jnp.float32).at[:Q].set(
        queue0.astype(jnp.float32))
    q1 = jnp.zeros((QPAD, C), jnp.float32).at[:Q].set(
        queue1.astype(jnp.float32))

    out = pl.pallas_call(
        functools.partial(_rca_kernel, nsteps=nsteps, num_p=P, num_q=Q,
                          batch=B),
        grid=(nsteps,),
        in_specs=[
            pl.BlockSpec((B, C, CHUNK), lambda j: (0, 0, j)),
            pl.BlockSpec((PPAD, C), lambda j: (0, 0)),
            pl.BlockSpec((C, PPAD), lambda j: (0, 0)),
            pl.BlockSpec((QPAD, C), lambda j: (0, 0)),
            pl.BlockSpec((QPAD, C), lambda j: (0, 0)),
        ],
        out_specs=pl.BlockSpec((1, 1), lambda j: (0, 0)),
        out_shape=jax.ShapeDtypeStruct((1, 1), jnp.float32),
        scratch_shapes=[pltpu.VMEM((B, PPAD, CHUNK), jnp.float32)],
    )(x2, pr, prT, q0, q1)
    return out.reshape(1)


# speedup vs baseline: 3.7072x; 1.1257x over previous
import functools

import jax
import jax.numpy as jnp
from jax.experimental import pallas as pl
from jax.experimental.pallas import tpu as pltpu

_TEMP = 0.07
_EPS = 1e-12


def _rca_kernel(x_ref, pr_ref, prT_ref, q0_ref, q1_ref, out_ref, acc_ref,
                *, nsteps, num_p, num_q, batch):
    j = pl.program_id(0)

    @pl.when(j == 0)
    def _init():
        acc_ref[...] = jnp.zeros_like(acc_ref)

    pr = pr_ref[...]
    for b in range(batch):
        xb = x_ref[b]
        sim = jax.lax.dot_general(pr, xb, (((1,), (0,)), ((), ())),
                                  preferred_element_type=jnp.float32)
        row = jax.lax.broadcasted_iota(jnp.int32, sim.shape, 0)
        sim = jnp.where(row < num_p, sim, -jnp.inf)
        mx = jnp.max(sim, axis=0, keepdims=True)
        idx = jnp.min(jnp.where(sim == mx, row, num_p), axis=0, keepdims=True)
        acc_ref[b] += (row == idx).astype(jnp.float32)

    @pl.when(j == nsteps - 1)
    def _loss():
        prT = prT_ref[...]
        q0 = q0_ref[...]
        q1 = q1_ref[...]
        rowq = jax.lax.broadcasted_iota(jnp.int32, (q0.shape[0], 1), 0)
        validq = rowq < num_q
        q0n = q0 / jnp.maximum(
            jnp.sqrt(jnp.sum(q0 * q0, axis=1, keepdims=True)), _EPS)
        q1n = q1 / jnp.maximum(
            jnp.sqrt(jnp.sum(q1 * q1, axis=1, keepdims=True)), _EPS)
        total = jnp.zeros((1, 1), jnp.float32)
        for b in range(batch):
            cnt = jnp.sum(acc_ref[b], axis=1, keepdims=True)
            pool = jax.lax.dot_general(prT, cnt, (((1,), (0,)), ((), ())),
                                       preferred_element_type=jnp.float32)
            n = pool / jnp.maximum(
                jnp.sqrt(jnp.sum(pool * pool, axis=0, keepdims=True)), _EPS)
            s_neg = jax.lax.dot_general(q0n, n, (((1,), (0,)), ((), ())),
                                        preferred_element_type=jnp.float32)
            logit_neg = jnp.where(validq, s_neg / _TEMP, -jnp.inf)
            m = jnp.max(logit_neg, axis=0, keepdims=True)
            eln = jnp.where(validq, jnp.exp(logit_neg - m), 0.0)
            l_neg = jnp.sum(eln, axis=0, keepdims=True)
            s_pos = jax.lax.dot_general(q1n, n, (((1,), (0,)), ((), ())),
                                        preferred_element_type=jnp.float32)
            logit_pos = s_pos / _TEMP - m
            elp = jnp.exp(logit_pos)
            terms = -(logit_pos - jnp.log(jnp.maximum(l_neg + elp, 1e-4)))
            loss_b = jnp.sum(jnp.where(validq, terms, 0.0),
                             axis=0, keepdims=True) / num_q
            total = total + loss_b
        out_ref[...] = total / batch


def kernel(x, label, prototypes, queue0, queue1):
    del label
    B, C = x.shape[0], x.shape[1]
    S = x.shape[2] * x.shape[3] * x.shape[4]
    P = prototypes.shape[0]
    Q = queue0.shape[0]
    PPAD = 16
    QPAD = 16
    CHUNK = 8192
    nsteps = S // CHUNK

    x2 = x.reshape(B, C, S)
    pr = jnp.zeros((PPAD, C), jnp.float32).at[:P].set(
        prototypes.reshape(P, C).astype(jnp.float32))
    prT = pr.T
    q0 = jnp.zeros((QPAD, C), jnp.float32).at[:Q].set(
        queue0.astype(jnp.float32))
    q1 = jnp.zeros((QPAD, C), jnp.float32).at[:Q].set(
        queue1.astype(jnp.float32))

    out = pl.pallas_call(
        functools.partial(_rca_kernel, nsteps=nsteps, num_p=P, num_q=Q,
                          batch=B),
        grid=(nsteps,),
        in_specs=[
            pl.BlockSpec((B, C, CHUNK), lambda j: (0, 0, j)),
            pl.BlockSpec((PPAD, C), lambda j: (0, 0)),
            pl.BlockSpec((C, PPAD), lambda j: (0, 0)),
            pl.BlockSpec((QPAD, C), lambda j: (0, 0)),
            pl.BlockSpec((QPAD, C), lambda j: (0, 0)),
        ],
        out_specs=pl.BlockSpec((1, 1), lambda j: (0, 0)),
        out_shape=jax.ShapeDtypeStruct((1, 1), jnp.float32),
        scratch_shapes=[pltpu.VMEM((B, PPAD, CHUNK), jnp.float32)],
    )(x2, pr, prT, q0, q1)
    return out.reshape(1)
